# baseline (speedup 1.0000x reference)
import functools

import jax
import jax.numpy as jnp
from jax import lax
from jax.experimental import pallas as pl
from jax.experimental.pallas import tpu as pltpu
from jax.experimental.pallas import tpu_sc as plsc

_N = 32768
_K = 2048
_ROWS = 128
_NW = 32
_RPW = _ROWS // _NW
_SL = _N // 16

_INT_MIN = -2147483648
_MANT = 0x7FFFFFFF
_Y_LO = 0x3F800000


def _sc_body(x_hbm, o_hbm, x_v, cand_y):
    wid = lax.axis_index("c") * 16 + lax.axis_index("s")
    lane = lax.iota(jnp.int32, 16)
    zero_f = jnp.zeros((16,), jnp.float32)

    def row_body(rr, _carry):
        row = wid * _RPW + rr
        pltpu.sync_copy(x_hbm.at[row], x_v)

        def make_compress(y_lo_static):
            def comp_body(j, off):
                x = x_v[pl.ds(j * 16, 16)]
                bits = lax.bitcast_convert_type(x, jnp.int32)
                y = jnp.where(bits >= 0, bits, bits ^ _MANT)
                m = y >= y_lo_static
                mi = m.astype(jnp.int32)
                pos = plsc.cumsum(mi)
                idx = pos + (off - 1)
                plsc.store_scatter(cand_y, [idx], y, mask=m)
                return off + jnp.sum(mi)
            return comp_body

        m_cnt = lax.fori_loop(0, _SL, make_compress(_Y_LO), jnp.int32(0))

        @pl.when(m_cnt < _K + 1)
        def _():
            lax.fori_loop(0, _SL, make_compress(_INT_MIN), 0)

        fell_back = m_cnt < _K + 1
        m_cnt2 = jnp.where(fell_back, jnp.int32(_N), m_cnt)
        y_floor = jnp.where(fell_back, jnp.int32(_INT_MIN), jnp.int32(_Y_LO))

        pad_idx = m_cnt2 + lane
        plsc.store_scatter(cand_y, [pad_idx],
                           jnp.full((16,), _MANT, jnp.int32),
                           mask=pad_idx < _N)

        r_a = m_cnt2 - (_K + 1)
        n_sl = (m_cnt2 + 15) // 16

        def bit_body(i, prefix):
            b = 31 - i
            c = prefix | lax.shift_left(jnp.int32(1), b)
            y_c = c ^ jnp.int32(_INT_MIN)

            def scan_count():
                y_c_vec = jnp.full((16,), y_c)

                def cnt_body(j, acc):
                    yv = cand_y[pl.ds(j * 16, 16)]
                    return acc + (yv < y_c_vec).astype(jnp.int32)

                acc = lax.fori_loop(0, n_sl, cnt_body,
                                    jnp.zeros((16,), jnp.int32))
                return jnp.sum(acc)

            cnt = lax.cond(y_c <= y_floor, lambda: jnp.int32(0), scan_count)
            return jnp.where(cnt <= r_a, c, prefix)

        prefix = lax.fori_loop(0, 32, bit_body, jnp.int32(0))

        y_t = jnp.full((16,), prefix ^ jnp.int32(_INT_MIN))
        bits_t = jnp.where(y_t >= 0, y_t, y_t ^ _MANT)
        thr = lax.bitcast_convert_type(bits_t, jnp.float32)

        def mask_body(j, carry):
            x = x_v[pl.ds(j * 16, 16)]
            x_v[pl.ds(j * 16, 16)] = jnp.where(x > thr, x, zero_f)
            return carry

        lax.fori_loop(0, _SL, mask_body, 0)
        pltpu.sync_copy(x_v, o_hbm.at[row])
        return _carry

    lax.fori_loop(0, _RPW, row_body, 0)


@jax.jit
def kernel(inputs):
    f = functools.partial(
        pl.kernel,
        out_type=jax.ShapeDtypeStruct((_ROWS, _N), jnp.float32),
        mesh=plsc.VectorSubcoreMesh(core_axis_name="c", subcore_axis_name="s"),
        scratch_types=[
            pltpu.VMEM((_N,), jnp.float32),
            pltpu.VMEM((_N,), jnp.int32),
        ],
        compiler_params=pltpu.CompilerParams(needs_layout_passes=False),
    )(_sc_body)
    return f(inputs)


# speedup vs baseline: 2.1564x; 2.1564x over previous
import functools

import jax
import jax.numpy as jnp
from jax import lax
from jax.experimental import pallas as pl
from jax.experimental.pallas import tpu as pltpu
from jax.experimental.pallas import tpu_sc as plsc

_N = 32768
_K = 2048
_ROWS = 128
_NW = 32
_RPW = _ROWS // _NW
_SL = _N // 16

_INT_MIN = -2147483648
_MANT = 0x7FFFFFFF
_Y_LO = 0x3FA66666
_UNROLL = 4


def _sc_body(x_hbm, o_hbm, x_v, cand_y):
    wid = lax.axis_index("c") * 16 + lax.axis_index("s")
    lane = lax.iota(jnp.int32, 16)
    zero_f = jnp.zeros((16,), jnp.float32)

    def row_body(rr, _carry):
        row = wid * _RPW + rr
        pltpu.sync_copy(x_hbm.at[row], x_v)

        def make_compress(y_lo_static):
            def comp_body(jj, off):
                for u in range(_UNROLL):
                    j = jj * _UNROLL + u
                    x = x_v[pl.ds(j * 16, 16)]
                    bits = lax.bitcast_convert_type(x, jnp.int32)
                    y = jnp.where(bits >= 0, bits, bits ^ _MANT)
                    m = y >= y_lo_static
                    plsc.store_compressed(
                        cand_y.at[pl.ds(jnp.minimum(off, _N - 16), 16)], y, mask=m)
                    pc = plsc.all_reduce_population_count(m)
                    off = off + pc[0]
                return off
            return comp_body

        m_cnt = lax.fori_loop(0, _SL // _UNROLL, make_compress(_Y_LO),
                              jnp.int32(0))

        need_fb = (m_cnt < _K + 1) | (m_cnt > _N - 16 * _UNROLL)

        @pl.when(need_fb)
        def _():
            lax.fori_loop(0, _SL // _UNROLL, make_compress(_INT_MIN), 0)

        fell_back = need_fb
        m_cnt2 = jnp.where(fell_back, jnp.int32(_N), m_cnt)
        y_floor = jnp.where(fell_back, jnp.int32(_INT_MIN), jnp.int32(_Y_LO))

        for u in range(_UNROLL):
            pad_idx = m_cnt2 + (16 * u) + lane
            plsc.store_scatter(cand_y, [pad_idx],
                               jnp.full((16,), _MANT, jnp.int32),
                               mask=pad_idx < _N)

        r_a = m_cnt2 - (_K + 1)
        n_bl = (m_cnt2 + 16 * _UNROLL - 1) // (16 * _UNROLL)

        def bit_body(i, prefix):
            b = 31 - i
            c = prefix | lax.shift_left(jnp.int32(1), b)
            y_c = c ^ jnp.int32(_INT_MIN)

            def scan_count():
                y_c_vec = jnp.full((16,), y_c)

                def cnt_body(jj, acc):
                    for u in range(_UNROLL):
                        yv = cand_y[pl.ds((jj * _UNROLL + u) * 16, 16)]
                        acc = acc + (yv < y_c_vec).astype(jnp.int32)
                    return acc

                acc = lax.fori_loop(0, n_bl, cnt_body,
                                    jnp.zeros((16,), jnp.int32))
                return jnp.sum(acc)

            cnt = lax.cond(y_c <= y_floor, lambda: jnp.int32(0), scan_count)
            return jnp.where(cnt <= r_a, c, prefix)

        prefix = lax.fori_loop(0, 32, bit_body, jnp.int32(0))

        y_t = jnp.full((16,), prefix ^ jnp.int32(_INT_MIN))
        bits_t = jnp.where(y_t >= 0, y_t, y_t ^ _MANT)
        thr = lax.bitcast_convert_type(bits_t, jnp.float32)

        def mask_body(jj, carry):
            for u in range(_UNROLL):
                j = jj * _UNROLL + u
                x = x_v[pl.ds(j * 16, 16)]
                x_v[pl.ds(j * 16, 16)] = jnp.where(x > thr, x, zero_f)
            return carry

        lax.fori_loop(0, _SL // _UNROLL, mask_body, 0)
        pltpu.sync_copy(x_v, o_hbm.at[row])
        return _carry

    lax.fori_loop(0, _RPW, row_body, 0)


@jax.jit
def kernel(inputs):
    f = functools.partial(
        pl.kernel,
        out_type=jax.ShapeDtypeStruct((_ROWS, _N), jnp.float32),
        mesh=plsc.VectorSubcoreMesh(core_axis_name="c", subcore_axis_name="s"),
        scratch_types=[
            pltpu.VMEM((_N,), jnp.float32),
            pltpu.VMEM((_N,), jnp.int32),
        ],
        compiler_params=pltpu.CompilerParams(needs_layout_passes=False),
    )(_sc_body)
    return f(inputs)


# speedup vs baseline: 4.3604x; 2.0221x over previous
import functools

import jax
import jax.numpy as jnp
from jax import lax
from jax.experimental import pallas as pl
from jax.experimental.pallas import tpu as pltpu
from jax.experimental.pallas import tpu_sc as plsc

_N = 32768
_K = 2048
_ROWS = 128
_NW = 32
_RPW = _ROWS // _NW
_SL = _N // 16

_INT_MIN = -2147483648
_MANT = 0x7FFFFFFF
_Y_LO = 0x3FA66666
_PIVOT = 1.3
_UNROLL = 8


def _sc_body(x_hbm, o_hbm, x_v, cand):
    wid = lax.axis_index("c") * 16 + lax.axis_index("s")
    lane = lax.iota(jnp.int32, 16)
    zero_f = jnp.zeros((16,), jnp.float32)

    def row_body(rr, _carry):
        row = wid * _RPW + rr
        pltpu.sync_copy(x_hbm.at[row], x_v)

        def make_compress(pivot):
            def comp_body(jj, off):
                xs, pcs = [], []
                for u in range(_UNROLL):
                    x = x_v[pl.ds((jj * _UNROLL + u) * 16, 16)]
                    m = x >= pivot
                    xs.append((x, m))
                    pcs.append(plsc.all_reduce_population_count(m)[0])
                for u in range(_UNROLL):
                    x, m = xs[u]
                    plsc.store_compressed(
                        cand.at[pl.ds(jnp.minimum(off, _N - 16), 16)],
                        x, mask=m)
                    off = off + pcs[u]
                return off
            return comp_body

        m_cnt = lax.fori_loop(0, _SL // _UNROLL, make_compress(_PIVOT),
                              jnp.int32(0))

        need_fb = (m_cnt < _K + 1) | (m_cnt > _N - 16 * _UNROLL)

        @pl.when(need_fb)
        def _():
            lax.fori_loop(0, _SL // _UNROLL, make_compress(-jnp.inf), 0)

        m_cnt2 = jnp.where(need_fb, jnp.int32(_N), m_cnt)
        y_floor = jnp.where(need_fb, jnp.int32(_INT_MIN), jnp.int32(_Y_LO))

        inf_f = jnp.full((16,), jnp.inf, jnp.float32)
        for u in range(_UNROLL):
            pad_idx = m_cnt2 + (16 * u) + lane
            plsc.store_scatter(cand, [pad_idx], inf_f, mask=pad_idx < _N)

        r_a = m_cnt2 - (_K + 1)
        n_bl = (m_cnt2 + 16 * _UNROLL - 1) // (16 * _UNROLL)

        def bit_body(i, prefix):
            b = 31 - i
            c = prefix | lax.shift_left(jnp.int32(1), b)
            y_c = c ^ jnp.int32(_INT_MIN)

            def scan_count():
                y_c_vec = jnp.full((16,), y_c)
                bits_c = jnp.where(y_c_vec >= 0, y_c_vec, y_c_vec ^ _MANT)
                piv = lax.bitcast_convert_type(bits_c, jnp.float32)

                def cnt_body(jj, accs):
                    a0, a1 = accs
                    for u in range(_UNROLL):
                        xv = cand[pl.ds((jj * _UNROLL + u) * 16, 16)]
                        hit = (xv < piv).astype(jnp.int32)
                        if u % 2 == 0:
                            a0 = a0 + hit
                        else:
                            a1 = a1 + hit
                    return a0, a1

                z16 = jnp.zeros((16,), jnp.int32)
                a0, a1 = lax.fori_loop(0, n_bl, cnt_body, (z16, z16))
                return jnp.sum(a0 + a1)

            cnt = lax.cond(y_c <= y_floor, lambda: jnp.int32(0), scan_count)
            return jnp.where(cnt <= r_a, c, prefix)

        prefix = lax.fori_loop(0, 32, bit_body, jnp.int32(0))

        y_t = jnp.full((16,), prefix ^ jnp.int32(_INT_MIN))
        bits_t = jnp.where(y_t >= 0, y_t, y_t ^ _MANT)
        thr = lax.bitcast_convert_type(bits_t, jnp.float32)

        def mask_body(jj, carry):
            for u in range(_UNROLL):
                j = jj * _UNROLL + u
                x = x_v[pl.ds(j * 16, 16)]
                x_v[pl.ds(j * 16, 16)] = jnp.where(x > thr, x, zero_f)
            return carry

        lax.fori_loop(0, _SL // _UNROLL, mask_body, 0)
        pltpu.sync_copy(x_v, o_hbm.at[row])
        return _carry

    lax.fori_loop(0, _RPW, row_body, 0)


@jax.jit
def kernel(inputs):
    f = functools.partial(
        pl.kernel,
        out_type=jax.ShapeDtypeStruct((_ROWS, _N), jnp.float32),
        mesh=plsc.VectorSubcoreMesh(core_axis_name="c", subcore_axis_name="s"),
        scratch_types=[
            pltpu.VMEM((_N,), jnp.float32),
            pltpu.VMEM((_N,), jnp.float32),
        ],
        compiler_params=pltpu.CompilerParams(needs_layout_passes=False),
    )(_sc_body)
    return f(inputs)


# speedup vs baseline: 4.4783x; 1.0270x over previous
import functools

import jax
import jax.numpy as jnp
from jax import lax
from jax.experimental import pallas as pl
from jax.experimental.pallas import tpu as pltpu
from jax.experimental.pallas import tpu_sc as plsc

_N = 32768
_K = 2048
_ROWS = 128
_NW = 32
_RPW = _ROWS // _NW
_SL = _N // 16

_INT_MIN = -2147483648
_MANT = 0x7FFFFFFF
_INF_BITS = 0x7F800000
_Y_LO = 0x3FB33333
_PIVOT = 1.4
_UNROLL = 8
_W2 = 2048
_B_SW = 12


def _decode(y_vec):
    bits = jnp.where(y_vec >= 0, y_vec, y_vec ^ _MANT)
    return lax.bitcast_convert_type(bits, jnp.float32)


def _sc_body(x_hbm, o_hbm, x_a, x_b, cand, cand2, sin_a, sin_b, sout_a,
             sout_b):
    wid = lax.axis_index("c") * 16 + lax.axis_index("s")
    lane = lax.iota(jnp.int32, 16)
    zero_f = jnp.zeros((16,), jnp.float32)
    inf_f = jnp.full((16,), jnp.inf, jnp.float32)

    bufs = [(x_a, sin_a, sout_a), (x_b, sin_b, sout_b)]
    row0 = wid * _RPW

    pltpu.async_copy(x_hbm.at[row0], x_a, sin_a)

    for rr in range(_RPW):
        x_v, sin, sout = bufs[rr % 2]
        nxt_v, nxt_in, nxt_out = bufs[(rr + 1) % 2]
        row = row0 + rr

        if rr + 1 < _RPW:
            if rr >= 1:
                pltpu.make_async_copy(nxt_v, o_hbm.at[row - 1], nxt_out).wait()
            pltpu.async_copy(x_hbm.at[row + 1], nxt_v, nxt_in)
        pltpu.make_async_copy(x_hbm.at[row], x_v, sin).wait()

        def make_compress(pivot):
            def comp_body(jj, off):
                xs, pcs = [], []
                for u in range(_UNROLL):
                    x = x_v[pl.ds((jj * _UNROLL + u) * 16, 16)]
                    m = x >= pivot
                    xs.append((x, m))
                    pcs.append(plsc.all_reduce_population_count(m)[0])
                for u in range(_UNROLL):
                    x, m = xs[u]
                    plsc.store_compressed(
                        cand.at[pl.ds(jnp.minimum(off, _N - 16), 16)],
                        x, mask=m)
                    off = off + pcs[u]
                return off
            return comp_body

        m_cnt = lax.fori_loop(0, _SL // _UNROLL, make_compress(_PIVOT),
                              jnp.int32(0))

        need_fb = (m_cnt < _K + 1) | (m_cnt > _N - 16 * _UNROLL)

        @pl.when(need_fb)
        def _():
            lax.fori_loop(0, _SL // _UNROLL, make_compress(-jnp.inf), 0)

        m_cnt2 = jnp.where(need_fb, jnp.int32(_N), m_cnt)
        y_floor = jnp.where(need_fb, jnp.int32(_INT_MIN), jnp.int32(_Y_LO))

        for u in range(_UNROLL):
            pad_idx = m_cnt2 + (16 * u) + lane
            plsc.store_scatter(cand, [pad_idx], inf_f, mask=pad_idx < _N)

        r_a = m_cnt2 - (_K + 1)
        n_bl = (m_cnt2 + 16 * _UNROLL - 1) // (16 * _UNROLL)

        def count_below(piv, n_blocks):
            def cnt_body(jj, accs):
                a0, a1 = accs
                for u in range(_UNROLL):
                    xv = cand[pl.ds((jj * _UNROLL + u) * 16, 16)]
                    hit = (xv < piv).astype(jnp.int32)
                    if u % 2 == 0:
                        a0 = a0 + hit
                    else:
                        a1 = a1 + hit
                return a0, a1

            z16 = jnp.zeros((16,), jnp.int32)
            a0, a1 = lax.fori_loop(0, n_blocks, cnt_body, (z16, z16))
            return jnp.sum(a0 + a1)

        def bit_body1(i, prefix):
            b = 31 - i
            c = prefix | lax.shift_left(jnp.int32(1), b)
            y_c = c ^ jnp.int32(_INT_MIN)
            cnt = lax.cond(
                y_c <= y_floor, lambda: jnp.int32(0),
                lambda: count_below(_decode(jnp.full((16,), y_c)), n_bl))
            return jnp.where(cnt <= r_a, c, prefix)

        prefix = lax.fori_loop(0, 32 - _B_SW, bit_body1, jnp.int32(0))

        lo_y = jnp.full((16,), prefix ^ jnp.int32(_INT_MIN))
        hi_y = jnp.minimum(lo_y + (1 << _B_SW), _INF_BITS)
        lo_f = _decode(lo_y)
        hi_f = _decode(hi_y)

        def win_body(jj, carry):
            off, c_lo = carry
            for u in range(_UNROLL):
                x = cand[pl.ds((jj * _UNROLL + u) * 16, 16)]
                m_lo = x < lo_f
                m_in = (x >= lo_f) & (x < hi_f)
                c_lo = c_lo + plsc.all_reduce_population_count(m_lo)[0]
                plsc.store_compressed(
                    cand2.at[pl.ds(jnp.minimum(off, _W2 - 16), 16)],
                    x, mask=m_in)
                off = off + plsc.all_reduce_population_count(m_in)[0]
            return off, c_lo

        w_cnt, c_lo = jnp.int32(0), jnp.int32(0)
        use2 = jnp.logical_not(need_fb)

        def run_window():
            return lax.fori_loop(0, n_bl, win_body,
                                 (jnp.int32(0), jnp.int32(0)))

        w_cnt, c_lo = lax.cond(
            use2, run_window, lambda: (jnp.int32(0), jnp.int32(0)))
        use2 = use2 & (w_cnt <= _W2 - 32)

        @pl.when(use2)
        def _():
            for u in range(2):
                pad_idx = w_cnt + (16 * u) + lane
                plsc.store_scatter(cand2, [pad_idx], inf_f,
                                   mask=pad_idx < _W2)

        n_bl2 = (w_cnt + 15) // 16

        def count2_below(piv):
            def cnt_body(jj, acc):
                xv = cand2[pl.ds(jj * 16, 16)]
                return acc + (xv < piv).astype(jnp.int32)
            acc = lax.fori_loop(0, n_bl2, cnt_body,
                                jnp.zeros((16,), jnp.int32))
            return jnp.sum(acc)

        def bit_body2(i, prefix):
            b = _B_SW - 1 - i
            c = prefix | lax.shift_left(jnp.int32(1), b)
            y_c = c ^ jnp.int32(_INT_MIN)
            piv = _decode(jnp.full((16,), y_c))
            cnt = lax.cond(
                use2,
                lambda: c_lo + count2_below(piv),
                lambda: lax.cond(
                    y_c <= y_floor, lambda: jnp.int32(0),
                    lambda: count_below(piv, n_bl)))
            return jnp.where(cnt <= r_a, c, prefix)

        prefix = lax.fori_loop(0, _B_SW, bit_body2, prefix)

        thr = _decode(jnp.full((16,), prefix ^ jnp.int32(_INT_MIN)))

        def mask_body(jj, carry):
            for u in range(_UNROLL):
                j = jj * _UNROLL + u
                x = x_v[pl.ds(j * 16, 16)]
                x_v[pl.ds(j * 16, 16)] = jnp.where(x > thr, x, zero_f)
            return carry

        lax.fori_loop(0, _SL // _UNROLL, mask_body, 0)
        pltpu.async_copy(x_v, o_hbm.at[row], sout)

    pltpu.make_async_copy(x_a, o_hbm.at[row0 + 2], sout_a).wait()
    pltpu.make_async_copy(x_b, o_hbm.at[row0 + 3], sout_b).wait()


@jax.jit
def kernel(inputs):
    f = functools.partial(
        pl.kernel,
        out_type=jax.ShapeDtypeStruct((_ROWS, _N), jnp.float32),
        mesh=plsc.VectorSubcoreMesh(core_axis_name="c", subcore_axis_name="s"),
        scratch_types=[
            pltpu.VMEM((_N,), jnp.float32),
            pltpu.VMEM((_N,), jnp.float32),
            pltpu.VMEM((_N,), jnp.float32),
            pltpu.VMEM((_W2,), jnp.float32),
            pltpu.SemaphoreType.DMA,
            pltpu.SemaphoreType.DMA,
            pltpu.SemaphoreType.DMA,
            pltpu.SemaphoreType.DMA,
        ],
        compiler_params=pltpu.CompilerParams(needs_layout_passes=False),
    )(_sc_body)
    return f(inputs)
